# Initial kernel scaffold; baseline (speedup 1.0000x reference)
import jax
import jax.numpy as jnp
from jax.experimental import pallas as pl


def kernel(energy, mol_idx):
    raise NotImplementedError("write your pallas kernel here")


# speedup vs baseline: 16.1813x; 16.1813x over previous
import functools

import jax
import jax.numpy as jnp
from jax import lax
from jax.experimental import pallas as pl
from jax.experimental.pallas import tpu as pltpu
from jax.experimental.pallas import tpu_sc as plsc

N = 6_400_000
S = 100_000
NC = 2
NS = 16
NW = NC * NS
CHUNK = N // NW
BLK = 1600
NBLK = CHUNK // BLK
VPB = BLK // 16
SEG_PAD = 100_352
ZSLICE = SEG_PAD // NS


def _sc_body(energy_hbm, idx_hbm, out_hbm, acc, e_buf, g_buf, idx_s, shared):
    c = lax.axis_index("c")
    s = lax.axis_index("s")
    wid = c * NS + s
    base = wid * CHUNK
    iota = lax.iota(jnp.int32, 16)
    zero_v = jnp.zeros((16,), jnp.float32)

    def zbody(j, _):
        acc[pl.ds(j * 16, 16)] = zero_v
        return 0
    lax.fori_loop(0, SEG_PAD // 16, zbody, 0)

    pltpu.sync_copy(acc.at[pl.ds(0, ZSLICE)], shared.at[pl.ds(s * ZSLICE, ZSLICE)])

    pltpu.sync_copy(idx_hbm.at[pl.ds(base, 16)], g_buf.at[pl.ds(0, 16)])
    pltpu.sync_copy(idx_hbm.at[pl.ds(base + CHUNK - 16, 16)], g_buf.at[pl.ds(16, 16)])
    first_id = g_buf[pl.ds(0, 16)][0]
    last_id = g_buf[pl.ds(16, 16)][15]

    def blk_body(b, _):
        off = base + b * BLK
        pltpu.sync_copy(energy_hbm.at[pl.ds(off, BLK)], e_buf)
        pltpu.sync_copy(idx_hbm.at[pl.ds(off, BLK)], g_buf)

        def vbody(j, _):
            v = e_buf[pl.ds(j * 16, 16)]
            g = g_buf[pl.ds(j * 16, 16)]
            plsc.addupdate_scatter(acc, [g], v)
            return 0
        lax.fori_loop(0, VPB, vbody, 0)
        return 0
    lax.fori_loop(0, NBLK, blk_body, 0)

    plsc.subcore_barrier()

    lo = (first_id // 8) * 8
    nblk2 = (last_id + 1 - lo + 127) // 128

    def comb_body(b, _):
        start = lo + b * 128
        for k in range(8):
            idx_s[pl.ds(k * 16, 16)] = start + k * 16 + iota
        pltpu.sync_copy(acc.at[pl.ds(start, 128)], shared.at[idx_s], add=True)
        return 0
    lax.fori_loop(0, nblk2, comb_body, 0)

    plsc.subcore_barrier()

    woff = s * ZSLICE
    pltpu.sync_copy(shared.at[pl.ds(woff, ZSLICE)], acc.at[pl.ds(0, ZSLICE)])
    pltpu.sync_copy(acc.at[pl.ds(0, ZSLICE)],
                    out_hbm.at[pl.ds(c * SEG_PAD + woff, ZSLICE)])


_sc_pool = pl.kernel(
    _sc_body,
    out_type=jax.ShapeDtypeStruct((NC * SEG_PAD,), jnp.float32),
    mesh=plsc.VectorSubcoreMesh(core_axis_name="c", subcore_axis_name="s"),
    compiler_params=pltpu.CompilerParams(
        use_tc_tiling_on_sc=False, needs_layout_passes=False),
    scratch_types=[
        pltpu.VMEM((SEG_PAD,), jnp.float32),
        pltpu.VMEM((BLK,), jnp.float32),
        pltpu.VMEM((BLK,), jnp.int32),
        pltpu.VMEM((128,), jnp.int32),
        pltpu.VMEM_SHARED((SEG_PAD,), jnp.float32),
    ],
)


def _add_body(p_ref, o_ref):
    o_ref[...] = p_ref[0] + p_ref[1]


_tc_add = pl.pallas_call(
    _add_body,
    out_shape=jax.ShapeDtypeStruct((SEG_PAD // 128, 128), jnp.float32),
)


def kernel(energy, mol_idx):
    partials = _sc_pool(energy, mol_idx)
    summed = _tc_add(partials.reshape(NC, SEG_PAD // 128, 128))
    return summed.reshape(-1)[:S]


# speedup vs baseline: 57.0797x; 3.5275x over previous
import functools

import jax
import jax.numpy as jnp
from jax import lax
from jax.experimental import pallas as pl
from jax.experimental.pallas import tpu as pltpu
from jax.experimental.pallas import tpu_sc as plsc

N = 6_400_000
S = 100_000
NC = 2
NS = 16
NW = NC * NS
CHUNK = N // NW
COL = 125
BLK = 16 * COL
NBLK = CHUNK // BLK
UNROLL = 5
SEG_PAD = 100_352
ZSLICE = SEG_PAD // NS


def _sc_body(energy_hbm, idx_hbm, out_hbm,
             acc, e0, g0, e1, g1, idx_s, sem0, sem1, shared):
    c = lax.axis_index("c")
    s = lax.axis_index("s")
    wid = c * NS + s
    base = wid * CHUNK
    iota = lax.iota(jnp.int32, 16)
    zero_v = jnp.zeros((16,), jnp.float32)

    def zbody(j, _):
        acc[pl.ds(j * 16, 16)] = zero_v
        return 0
    lax.fori_loop(0, SEG_PAD // 16, zbody, 0)

    pltpu.sync_copy(acc.at[pl.ds(0, ZSLICE)], shared.at[pl.ds(s * ZSLICE, ZSLICE)])

    pltpu.sync_copy(idx_hbm.at[pl.ds(base, 16)], g0.at[pl.ds(0, 16)])
    pltpu.sync_copy(idx_hbm.at[pl.ds(base + CHUNK - 16, 16)], g0.at[pl.ds(16, 16)])
    first_id = g0[pl.ds(0, 16)][0]
    last_id = g0[pl.ds(16, 16)][15]

    def start_blk(b, eb, gb, sem):
        off = base + b * BLK
        pltpu.async_copy(energy_hbm.at[pl.ds(off, BLK)], eb, sem)
        pltpu.async_copy(idx_hbm.at[pl.ds(off, BLK)], gb, sem)

    def wait_blk(b, eb, gb, sem):
        off = base + b * BLK
        pltpu.make_async_copy(energy_hbm.at[pl.ds(off, BLK)], eb, sem).wait()
        pltpu.make_async_copy(idx_hbm.at[pl.ds(off, BLK)], gb, sem).wait()

    col_iota = iota * COL

    def process(eb, gb):
        def vbody(k, idxv):
            for _ in range(UNROLL):
                v = plsc.load_gather(eb, [idxv])
                g = plsc.load_gather(gb, [idxv])
                plsc.addupdate_scatter(acc, [g], v)
                idxv = idxv + 1
            return idxv
        lax.fori_loop(0, COL // UNROLL, vbody, col_iota)

    start_blk(0, e0, g0, sem0)

    def blk_pair(p, _):
        b0 = p * 2
        start_blk(b0 + 1, e1, g1, sem1)
        wait_blk(b0, e0, g0, sem0)
        process(e0, g0)

        @pl.when(b0 + 2 < NBLK)
        def _():
            start_blk(b0 + 2, e0, g0, sem0)
        wait_blk(b0 + 1, e1, g1, sem1)
        process(e1, g1)
        return 0
    lax.fori_loop(0, NBLK // 2, blk_pair, 0)

    plsc.subcore_barrier()

    lo = (first_id // 8) * 8
    nblk2 = (last_id + 1 - lo + 127) // 128

    def comb_body(b, _):
        start = lo + b * 128
        for k in range(8):
            idx_s[pl.ds(k * 16, 16)] = start + k * 16 + iota
        pltpu.sync_copy(acc.at[pl.ds(start, 128)], shared.at[idx_s], add=True)
        return 0
    lax.fori_loop(0, nblk2, comb_body, 0)

    plsc.subcore_barrier()

    woff = s * ZSLICE
    pltpu.sync_copy(shared.at[pl.ds(woff, ZSLICE)], acc.at[pl.ds(0, ZSLICE)])
    pltpu.sync_copy(acc.at[pl.ds(0, ZSLICE)],
                    out_hbm.at[pl.ds(c * SEG_PAD + woff, ZSLICE)])


_sc_pool = pl.kernel(
    _sc_body,
    out_type=jax.ShapeDtypeStruct((NC * SEG_PAD,), jnp.float32),
    mesh=plsc.VectorSubcoreMesh(core_axis_name="c", subcore_axis_name="s"),
    compiler_params=pltpu.CompilerParams(
        use_tc_tiling_on_sc=False, needs_layout_passes=False),
    scratch_types=[
        pltpu.VMEM((SEG_PAD,), jnp.float32),
        pltpu.VMEM((BLK,), jnp.float32),
        pltpu.VMEM((BLK,), jnp.int32),
        pltpu.VMEM((BLK,), jnp.float32),
        pltpu.VMEM((BLK,), jnp.int32),
        pltpu.VMEM((128,), jnp.int32),
        pltpu.SemaphoreType.DMA,
        pltpu.SemaphoreType.DMA,
        pltpu.VMEM_SHARED((SEG_PAD,), jnp.float32),
    ],
)


def _add_body(p_ref, o_ref):
    o_ref[...] = p_ref[0] + p_ref[1]


_tc_add = pl.pallas_call(
    _add_body,
    out_shape=jax.ShapeDtypeStruct((SEG_PAD // 128, 128), jnp.float32),
)


def kernel(energy, mol_idx):
    partials = _sc_pool(energy, mol_idx)
    summed = _tc_add(partials.reshape(NC, SEG_PAD // 128, 128))
    return summed.reshape(-1)[:S]


# speedup vs baseline: 57.1335x; 1.0009x over previous
import functools

import jax
import jax.numpy as jnp
from jax import lax
from jax.experimental import pallas as pl
from jax.experimental.pallas import tpu as pltpu
from jax.experimental.pallas import tpu_sc as plsc

N = 6_400_000
S = 100_000
NC = 2
NS = 16
NW = NC * NS
CHUNK = N // NW
COL = 125
BLK = 16 * COL
NBLK = CHUNK // BLK
UNROLL = 25
SEG_PAD = 100_352
ZSLICE = SEG_PAD // NS


def _sc_body(energy_hbm, idx_hbm, out_hbm,
             acc, e0, g0, e1, g1, idx_s, sem0, sem1, shared):
    c = lax.axis_index("c")
    s = lax.axis_index("s")
    wid = c * NS + s
    base = wid * CHUNK
    iota = lax.iota(jnp.int32, 16)
    zero_v = jnp.zeros((16,), jnp.float32)

    def zbody(j, _):
        acc[pl.ds(j * 16, 16)] = zero_v
        return 0
    lax.fori_loop(0, SEG_PAD // 16, zbody, 0)

    pltpu.sync_copy(acc.at[pl.ds(0, ZSLICE)], shared.at[pl.ds(s * ZSLICE, ZSLICE)])

    pltpu.sync_copy(idx_hbm.at[pl.ds(base, 16)], g0.at[pl.ds(0, 16)])
    pltpu.sync_copy(idx_hbm.at[pl.ds(base + CHUNK - 16, 16)], g0.at[pl.ds(16, 16)])
    first_id = g0[pl.ds(0, 16)][0]
    last_id = g0[pl.ds(16, 16)][15]

    def start_blk(b, eb, gb, sem):
        off = base + b * BLK
        pltpu.async_copy(energy_hbm.at[pl.ds(off, BLK)], eb, sem)
        pltpu.async_copy(idx_hbm.at[pl.ds(off, BLK)], gb, sem)

    def wait_blk(b, eb, gb, sem):
        off = base + b * BLK
        pltpu.make_async_copy(energy_hbm.at[pl.ds(off, BLK)], eb, sem).wait()
        pltpu.make_async_copy(idx_hbm.at[pl.ds(off, BLK)], gb, sem).wait()

    col_iota = iota * COL

    def process(eb, gb):
        def vbody(k, idxv):
            for _ in range(UNROLL):
                v = plsc.load_gather(eb, [idxv])
                g = plsc.load_gather(gb, [idxv])
                plsc.addupdate_scatter(acc, [g], v)
                idxv = idxv + 1
            return idxv
        lax.fori_loop(0, COL // UNROLL, vbody, col_iota)

    start_blk(0, e0, g0, sem0)

    def blk_pair(p, _):
        b0 = p * 2
        start_blk(b0 + 1, e1, g1, sem1)
        wait_blk(b0, e0, g0, sem0)
        process(e0, g0)

        @pl.when(b0 + 2 < NBLK)
        def _():
            start_blk(b0 + 2, e0, g0, sem0)
        wait_blk(b0 + 1, e1, g1, sem1)
        process(e1, g1)
        return 0
    lax.fori_loop(0, NBLK // 2, blk_pair, 0)

    plsc.subcore_barrier()

    lo = (first_id // 8) * 8
    nblk2 = (last_id + 1 - lo + 127) // 128

    def comb_body(b, _):
        start = lo + b * 128
        for k in range(8):
            idx_s[pl.ds(k * 16, 16)] = start + k * 16 + iota
        pltpu.sync_copy(acc.at[pl.ds(start, 128)], shared.at[idx_s], add=True)
        return 0
    lax.fori_loop(0, nblk2, comb_body, 0)

    plsc.subcore_barrier()

    woff = s * ZSLICE
    pltpu.sync_copy(shared.at[pl.ds(woff, ZSLICE)], acc.at[pl.ds(0, ZSLICE)])
    pltpu.sync_copy(acc.at[pl.ds(0, ZSLICE)],
                    out_hbm.at[pl.ds(c * SEG_PAD + woff, ZSLICE)])


_sc_pool = pl.kernel(
    _sc_body,
    out_type=jax.ShapeDtypeStruct((NC * SEG_PAD,), jnp.float32),
    mesh=plsc.VectorSubcoreMesh(core_axis_name="c", subcore_axis_name="s"),
    compiler_params=pltpu.CompilerParams(
        use_tc_tiling_on_sc=False, needs_layout_passes=False),
    scratch_types=[
        pltpu.VMEM((SEG_PAD,), jnp.float32),
        pltpu.VMEM((BLK,), jnp.float32),
        pltpu.VMEM((BLK,), jnp.int32),
        pltpu.VMEM((BLK,), jnp.float32),
        pltpu.VMEM((BLK,), jnp.int32),
        pltpu.VMEM((128,), jnp.int32),
        pltpu.SemaphoreType.DMA,
        pltpu.SemaphoreType.DMA,
        pltpu.VMEM_SHARED((SEG_PAD,), jnp.float32),
    ],
)


def _add_body(p_ref, o_ref):
    o_ref[...] = p_ref[0] + p_ref[1]


_tc_add = pl.pallas_call(
    _add_body,
    out_shape=jax.ShapeDtypeStruct((SEG_PAD // 128, 128), jnp.float32),
)


def kernel(energy, mol_idx):
    partials = _sc_pool(energy, mol_idx)
    summed = _tc_add(partials.reshape(NC, SEG_PAD // 128, 128))
    return summed.reshape(-1)[:S]


# speedup vs baseline: 59.8305x; 1.0472x over previous
import functools

import jax
import jax.numpy as jnp
from jax import lax
from jax.experimental import pallas as pl
from jax.experimental.pallas import tpu as pltpu
from jax.experimental.pallas import tpu_sc as plsc

N = 6_400_000
S = 100_000
NC = 2
NS = 16
NW = NC * NS
CHUNK = N // NW
COL = 125
SUB = 16 * COL
SPB = 2
BLK = SUB * SPB
NBLK = CHUNK // BLK
UNROLL = 25
SEG_PAD = 100_352
ZSLICE = SEG_PAD // NS


def _sc_body(energy_hbm, idx_hbm, out_hbm,
             acc, e0, g0, e1, g1, idx_s, sem0, sem1, shared):
    c = lax.axis_index("c")
    s = lax.axis_index("s")
    wid = c * NS + s
    base = wid * CHUNK
    iota = lax.iota(jnp.int32, 16)
    zero_v = jnp.zeros((16,), jnp.float32)

    def zbody(j, _):
        acc[pl.ds(j * 16, 16)] = zero_v
        return 0
    lax.fori_loop(0, SEG_PAD // 16, zbody, 0)

    pltpu.sync_copy(acc.at[pl.ds(0, ZSLICE)], shared.at[pl.ds(s * ZSLICE, ZSLICE)])

    pltpu.sync_copy(idx_hbm.at[pl.ds(base, 16)], g0.at[pl.ds(0, 16)])
    pltpu.sync_copy(idx_hbm.at[pl.ds(base + CHUNK - 16, 16)], g0.at[pl.ds(16, 16)])
    first_id = g0[pl.ds(0, 16)][0]
    last_id = g0[pl.ds(16, 16)][15]

    def start_blk(b, eb, gb, sem):
        off = base + b * BLK
        pltpu.async_copy(energy_hbm.at[pl.ds(off, BLK)], eb, sem)
        pltpu.async_copy(idx_hbm.at[pl.ds(off, BLK)], gb, sem)

    def wait_blk(b, eb, gb, sem):
        off = base + b * BLK
        pltpu.make_async_copy(energy_hbm.at[pl.ds(off, BLK)], eb, sem).wait()
        pltpu.make_async_copy(idx_hbm.at[pl.ds(off, BLK)], gb, sem).wait()

    col_iota = iota * COL

    def process(eb, gb):
        for sub in range(SPB):
            def vbody(k, idxv):
                for _ in range(UNROLL):
                    v = plsc.load_gather(eb, [idxv])
                    g = plsc.load_gather(gb, [idxv])
                    plsc.addupdate_scatter(acc, [g], v)
                    idxv = idxv + 1
                return idxv
            lax.fori_loop(0, COL // UNROLL, vbody, col_iota + sub * SUB)

    start_blk(0, e0, g0, sem0)

    def blk_pair(p, _):
        b0 = p * 2
        start_blk(b0 + 1, e1, g1, sem1)
        wait_blk(b0, e0, g0, sem0)
        process(e0, g0)

        @pl.when(b0 + 2 < NBLK)
        def _():
            start_blk(b0 + 2, e0, g0, sem0)
        wait_blk(b0 + 1, e1, g1, sem1)
        process(e1, g1)
        return 0
    lax.fori_loop(0, NBLK // 2, blk_pair, 0)

    plsc.subcore_barrier()

    lo = (first_id // 8) * 8
    nblk2 = (last_id + 1 - lo + 127) // 128

    def comb_body(b, _):
        start = lo + b * 128
        for k in range(8):
            idx_s[pl.ds(k * 16, 16)] = start + k * 16 + iota
        pltpu.sync_copy(acc.at[pl.ds(start, 128)], shared.at[idx_s], add=True)
        return 0
    lax.fori_loop(0, nblk2, comb_body, 0)

    plsc.subcore_barrier()

    woff = s * ZSLICE
    pltpu.sync_copy(shared.at[pl.ds(woff, ZSLICE)], acc.at[pl.ds(0, ZSLICE)])
    pltpu.sync_copy(acc.at[pl.ds(0, ZSLICE)],
                    out_hbm.at[pl.ds(c * SEG_PAD + woff, ZSLICE)])


_sc_pool = pl.kernel(
    _sc_body,
    out_type=jax.ShapeDtypeStruct((NC * SEG_PAD,), jnp.float32),
    mesh=plsc.VectorSubcoreMesh(core_axis_name="c", subcore_axis_name="s"),
    compiler_params=pltpu.CompilerParams(
        use_tc_tiling_on_sc=False, needs_layout_passes=False),
    scratch_types=[
        pltpu.VMEM((SEG_PAD,), jnp.float32),
        pltpu.VMEM((BLK,), jnp.float32),
        pltpu.VMEM((BLK,), jnp.int32),
        pltpu.VMEM((BLK,), jnp.float32),
        pltpu.VMEM((BLK,), jnp.int32),
        pltpu.VMEM((128,), jnp.int32),
        pltpu.SemaphoreType.DMA,
        pltpu.SemaphoreType.DMA,
        pltpu.VMEM_SHARED((SEG_PAD,), jnp.float32),
    ],
)


def _add_body(p_ref, o_ref):
    o_ref[...] = p_ref[0] + p_ref[1]


_tc_add = pl.pallas_call(
    _add_body,
    out_shape=jax.ShapeDtypeStruct((SEG_PAD // 128, 128), jnp.float32),
)


def kernel(energy, mol_idx):
    partials = _sc_pool(energy, mol_idx)
    summed = _tc_add(partials.reshape(NC, SEG_PAD // 128, 128))
    return summed.reshape(-1)[:S]


# speedup vs baseline: 62.5029x; 1.0447x over previous
import functools

import jax
import jax.numpy as jnp
from jax import lax
from jax.experimental import pallas as pl
from jax.experimental.pallas import tpu as pltpu
from jax.experimental.pallas import tpu_sc as plsc

N = 6_400_000
S = 100_000
NC = 2
NS = 16
NW = NC * NS
CHUNK = N // NW
COL = 125
SUB = 16 * COL
SPB = 2
BLK = SUB * SPB
NBLK = CHUNK // BLK
UNROLL = 31
SEG_PAD = 100_352
ZSLICE = SEG_PAD // NS


def _sc_body(energy_hbm, idx_hbm, out_hbm,
             acc, e0, g0, e1, g1, idx_s, sem0, sem1, shared):
    c = lax.axis_index("c")
    s = lax.axis_index("s")
    wid = c * NS + s
    base = wid * CHUNK
    iota = lax.iota(jnp.int32, 16)
    zero_v = jnp.zeros((16,), jnp.float32)

    pltpu.sync_copy(idx_hbm.at[pl.ds(base, 16)], g0.at[pl.ds(0, 16)])
    pltpu.sync_copy(idx_hbm.at[pl.ds(base + CHUNK - 16, 16)], g0.at[pl.ds(16, 16)])
    first_id = g0[pl.ds(0, 16)][0]
    last_id = g0[pl.ds(16, 16)][15]

    lo = (first_id // 8) * 8
    nblk2 = (last_id + 1 - lo + 127) // 128

    def zbody(j, _):
        acc[pl.ds(lo + j * 16, 16)] = zero_v
        return 0
    lax.fori_loop(0, nblk2 * 8, zbody, 0)

    def zs1(j, _):
        e0[pl.ds(j * 16, 16)] = zero_v
        return 0
    lax.fori_loop(0, 128, zs1, 0)
    for j in range(ZSLICE // 2048):
        pltpu.sync_copy(e0.at[pl.ds(0, 2048)],
                        shared.at[pl.ds(s * ZSLICE + j * 2048, 2048)])
    pltpu.sync_copy(e0.at[pl.ds(0, ZSLICE % 2048)],
                    shared.at[pl.ds(s * ZSLICE + (ZSLICE // 2048) * 2048,
                                    ZSLICE % 2048)])

    def start_blk(b, eb, gb, sem):
        off = base + b * BLK
        pltpu.async_copy(energy_hbm.at[pl.ds(off, BLK)], eb, sem)
        pltpu.async_copy(idx_hbm.at[pl.ds(off, BLK)], gb, sem)

    def wait_blk(b, eb, gb, sem):
        off = base + b * BLK
        pltpu.make_async_copy(energy_hbm.at[pl.ds(off, BLK)], eb, sem).wait()
        pltpu.make_async_copy(idx_hbm.at[pl.ds(off, BLK)], gb, sem).wait()

    col_iota = iota * COL

    def process(eb, gb):
        for sub in range(SPB):
            base_iota = col_iota + sub * SUB
            cur_id0 = plsc.load_gather(gb, [base_iota])
            cur_sum0 = plsc.load_gather(eb, [base_iota])

            def vbody(k, carry):
                idxv, cur_id, cur_sum = carry
                for _ in range(UNROLL):
                    idxv = idxv + 1
                    g = plsc.load_gather(gb, [idxv])
                    v = plsc.load_gather(eb, [idxv])
                    changed = g != cur_id
                    plsc.addupdate_scatter(acc, [cur_id], cur_sum, mask=changed)
                    cur_sum = jnp.where(changed, v, cur_sum + v)
                    cur_id = g
                return (idxv, cur_id, cur_sum)
            _, cur_id, cur_sum = lax.fori_loop(
                0, (COL - 1) // UNROLL, vbody, (base_iota, cur_id0, cur_sum0))
            plsc.addupdate_scatter(acc, [cur_id], cur_sum)

    start_blk(0, e0, g0, sem0)

    def blk_pair(p, _):
        b0 = p * 2
        start_blk(b0 + 1, e1, g1, sem1)
        wait_blk(b0, e0, g0, sem0)
        process(e0, g0)

        @pl.when(b0 + 2 < NBLK)
        def _():
            start_blk(b0 + 2, e0, g0, sem0)
        wait_blk(b0 + 1, e1, g1, sem1)
        process(e1, g1)
        return 0
    lax.fori_loop(0, NBLK // 2, blk_pair, 0)

    plsc.subcore_barrier()

    def comb_body(b, _):
        start = lo + b * 128
        for k in range(8):
            idx_s[pl.ds(k * 16, 16)] = start + k * 16 + iota
        pltpu.sync_copy(acc.at[pl.ds(start, 128)], shared.at[idx_s], add=True)
        return 0
    lax.fori_loop(0, nblk2, comb_body, 0)

    plsc.subcore_barrier()

    woff = s * ZSLICE
    pltpu.sync_copy(shared.at[pl.ds(woff, ZSLICE)], acc.at[pl.ds(0, ZSLICE)])
    pltpu.sync_copy(acc.at[pl.ds(0, ZSLICE)],
                    out_hbm.at[pl.ds(c * SEG_PAD + woff, ZSLICE)])


_sc_pool = pl.kernel(
    _sc_body,
    out_type=jax.ShapeDtypeStruct((NC * SEG_PAD,), jnp.float32),
    mesh=plsc.VectorSubcoreMesh(core_axis_name="c", subcore_axis_name="s"),
    compiler_params=pltpu.CompilerParams(
        use_tc_tiling_on_sc=False, needs_layout_passes=False),
    scratch_types=[
        pltpu.VMEM((SEG_PAD,), jnp.float32),
        pltpu.VMEM((BLK,), jnp.float32),
        pltpu.VMEM((BLK,), jnp.int32),
        pltpu.VMEM((BLK,), jnp.float32),
        pltpu.VMEM((BLK,), jnp.int32),
        pltpu.VMEM((128,), jnp.int32),
        pltpu.SemaphoreType.DMA,
        pltpu.SemaphoreType.DMA,
        pltpu.VMEM_SHARED((SEG_PAD,), jnp.float32),
    ],
)


def _add_body(p_ref, o_ref):
    o_ref[...] = p_ref[0] + p_ref[1]


_tc_add = pl.pallas_call(
    _add_body,
    out_shape=jax.ShapeDtypeStruct((SEG_PAD // 128, 128), jnp.float32),
)


def kernel(energy, mol_idx):
    partials = _sc_pool(energy, mol_idx)
    summed = _tc_add(partials.reshape(NC, SEG_PAD // 128, 128))
    return summed.reshape(-1)[:S]
